# Initial kernel scaffold; baseline (speedup 1.0000x reference)
import jax
import jax.numpy as jnp
from jax.experimental import pallas as pl


def kernel(ss, ps, os, ent_embedding, rel_embedding):
    raise NotImplementedError("write your pallas kernel here")


# speedup vs baseline: 1.6863x; 1.6863x over previous
import functools

import jax
import jax.numpy as jnp
from jax import lax
from jax.experimental import pallas as pl
from jax.experimental.pallas import tpu as pltpu
from jax.experimental.pallas import tpu_sc as plsc

_EMBED = 128
_NUM_WORKERS = 32
_CHUNK = 128


def _sc_gather(ss, ps, os_idx, ent, rel):
    batch = ss.shape[0]
    b_per_w = batch // _NUM_WORKERS
    rows_t = jax.ShapeDtypeStruct((batch, _EMBED), jnp.float32)
    mesh = plsc.VectorSubcoreMesh(core_axis_name="c", subcore_axis_name="s")

    @functools.partial(
        pl.kernel,
        out_type=[rows_t, rows_t, rows_t],
        mesh=mesh,
        scratch_types=[
            pltpu.VMEM((_CHUNK,), jnp.int32),
            pltpu.VMEM((_CHUNK, _EMBED), jnp.float32),
            pltpu.SemaphoreType.DMA,
        ],
    )
    def gather_kernel(ss_hbm, ps_hbm, os_hbm, ent_hbm, rel_hbm,
                      s_out, p_out, o_out, idx_v, rows_v, sem):
        wid = lax.axis_index("s") * 2 + lax.axis_index("c")
        base = wid * b_per_w
        for idx_hbm, table_hbm, out_hbm in (
            (ss_hbm, ent_hbm, s_out),
            (ps_hbm, rel_hbm, p_out),
            (os_hbm, ent_hbm, o_out),
        ):
            @pl.loop(0, b_per_w, step=_CHUNK)
            def _(c, idx_hbm=idx_hbm, table_hbm=table_hbm, out_hbm=out_hbm):
                pltpu.sync_copy(idx_hbm.at[pl.ds(base + c, _CHUNK)], idx_v)
                pltpu.async_copy(table_hbm.at[idx_v], rows_v, sem).wait()
                pltpu.sync_copy(rows_v, out_hbm.at[pl.ds(base + c, _CHUNK)])

    return gather_kernel(ss, ps, os_idx, ent, rel)


def _score_block(s_ref, p_ref, o_ref, out_ref):
    def _norm(x):
        n = jnp.sqrt(jnp.sum(x * x, axis=-1, keepdims=True))
        return x / jnp.maximum(n, 1e-12)

    d = (_norm(o_ref[...]) - _norm(s_ref[...])) - _norm(p_ref[...]) + 1e-6
    out_ref[...] = jnp.sqrt(jnp.sum(d * d, axis=-1))


def _tc_score(s_rows, p_rows, o_rows):
    batch = s_rows.shape[0]
    blk = 2048
    row_spec = pl.BlockSpec((blk, _EMBED), lambda i: (i, 0))
    return pl.pallas_call(
        _score_block,
        grid=(batch // blk,),
        in_specs=[row_spec, row_spec, row_spec],
        out_specs=pl.BlockSpec((blk,), lambda i: (i,)),
        out_shape=jax.ShapeDtypeStruct((batch,), jnp.float32),
    )(s_rows, p_rows, o_rows)


def kernel(ss, ps, os, ent_embedding, rel_embedding):
    ss = ss.astype(jnp.int32)
    ps = ps.astype(jnp.int32)
    os_idx = os.astype(jnp.int32)
    s_rows, p_rows, o_rows = _sc_gather(ss, ps, os_idx,
                                        ent_embedding, rel_embedding)
    return _tc_score(s_rows, p_rows, o_rows)


# speedup vs baseline: 1.9483x; 1.1553x over previous
import functools

import jax
import jax.numpy as jnp
from jax import lax
from jax.experimental import pallas as pl
from jax.experimental.pallas import tpu as pltpu
from jax.experimental.pallas import tpu_sc as plsc

_EMBED = 128
_NUM_WORKERS = 32
_CHUNK = 128


_NBUF = 4


def _sc_gather(ss, ps, os_idx, ent, rel):
    batch = ss.shape[0]
    b_per_w = batch // _NUM_WORKERS
    rows_t = jax.ShapeDtypeStruct((batch, _EMBED), jnp.float32)
    mesh = plsc.VectorSubcoreMesh(core_axis_name="c", subcore_axis_name="s")

    @functools.partial(
        pl.kernel,
        out_type=[rows_t, rows_t, rows_t],
        mesh=mesh,
        scratch_types=(
            [pltpu.VMEM((b_per_w,), jnp.int32) for _ in range(3)]
            + [pltpu.VMEM((_CHUNK, _EMBED), jnp.float32) for _ in range(_NBUF)]
            + [pltpu.SemaphoreType.DMA for _ in range(2 * _NBUF + 1)]
        ),
    )
    def gather_kernel(ss_hbm, ps_hbm, os_hbm, ent_hbm, rel_hbm,
                      s_out, p_out, o_out, sidx, pidx, oidx, *rest):
        bufs = rest[:_NBUF]
        gsem = rest[_NBUF:2 * _NBUF]
        wsem = rest[2 * _NBUF:3 * _NBUF]
        isem = rest[3 * _NBUF]
        wid = lax.axis_index("s") * 2 + lax.axis_index("c")
        base = wid * b_per_w

        ih = [pltpu.async_copy(src.at[pl.ds(base, b_per_w)], dst, isem)
              for src, dst in ((ss_hbm, sidx), (ps_hbm, pidx), (os_hbm, oidx))]
        for h in ih:
            h.wait()

        items = []
        for idxr, tab, out in ((sidx, ent_hbm, s_out),
                               (pidx, rel_hbm, p_out),
                               (oidx, ent_hbm, o_out)):
            for c in range(0, b_per_w, _CHUNK):
                items.append((idxr, c, tab, out))

        g_h = [None] * _NBUF
        w_h = [None] * _NBUF
        prev = None

        def _start_writeback(i, j):
            _, off, _, out = items[i]
            g_h[j].wait()
            w_h[j] = pltpu.async_copy(
                bufs[j], out.at[pl.ds(base + off, _CHUNK)], wsem[j])

        for i, (idxr, off, tab, _) in enumerate(items):
            j = i % _NBUF
            if w_h[j] is not None:
                w_h[j].wait()
                w_h[j] = None
            g_h[j] = pltpu.async_copy(
                tab.at[idxr.at[pl.ds(off, _CHUNK)]], bufs[j], gsem[j])
            if prev is not None:
                _start_writeback(*prev)
            prev = (i, j)
        _start_writeback(*prev)
        for j in range(_NBUF):
            if w_h[j] is not None:
                w_h[j].wait()

    return gather_kernel(ss, ps, os_idx, ent, rel)


def _score_block(s_ref, p_ref, o_ref, out_ref):
    def _norm(x):
        n = jnp.sqrt(jnp.sum(x * x, axis=-1, keepdims=True))
        return x / jnp.maximum(n, 1e-12)

    d = (_norm(o_ref[...]) - _norm(s_ref[...])) - _norm(p_ref[...]) + 1e-6
    out_ref[...] = jnp.sqrt(jnp.sum(d * d, axis=-1))


def _tc_score(s_rows, p_rows, o_rows):
    batch = s_rows.shape[0]
    blk = 2048
    row_spec = pl.BlockSpec((blk, _EMBED), lambda i: (i, 0))
    return pl.pallas_call(
        _score_block,
        grid=(batch // blk,),
        in_specs=[row_spec, row_spec, row_spec],
        out_specs=pl.BlockSpec((blk,), lambda i: (i,)),
        out_shape=jax.ShapeDtypeStruct((batch,), jnp.float32),
    )(s_rows, p_rows, o_rows)


def kernel(ss, ps, os, ent_embedding, rel_embedding):
    ss = ss.astype(jnp.int32)
    ps = ps.astype(jnp.int32)
    os_idx = os.astype(jnp.int32)
    s_rows, p_rows, o_rows = _sc_gather(ss, ps, os_idx,
                                        ent_embedding, rel_embedding)
    return _tc_score(s_rows, p_rows, o_rows)


# speedup vs baseline: 1.9856x; 1.0192x over previous
import functools

import jax
import jax.numpy as jnp
from jax import lax
from jax.experimental import pallas as pl
from jax.experimental.pallas import tpu as pltpu
from jax.experimental.pallas import tpu_sc as plsc

_EMBED = 128
_NUM_WORKERS = 32
_CHUNK = 128


_NBUF = 4


def _sc_gather(ss, ps, os_idx, ent, rel):
    batch = ss.shape[0]
    b_per_w = batch // _NUM_WORKERS
    rows_t = jax.ShapeDtypeStruct((batch, _EMBED), jnp.float32)
    mesh = plsc.VectorSubcoreMesh(core_axis_name="c", subcore_axis_name="s")

    @functools.partial(
        pl.kernel,
        out_type=[rows_t, rows_t, rows_t],
        mesh=mesh,
        scratch_types=(
            [pltpu.VMEM((b_per_w,), jnp.int32) for _ in range(3)]
            + [pltpu.VMEM((_CHUNK, _EMBED), jnp.float32) for _ in range(_NBUF)]
            + [pltpu.SemaphoreType.DMA for _ in range(2 * _NBUF + 1)]
        ),
    )
    def gather_kernel(ss_hbm, ps_hbm, os_hbm, ent_hbm, rel_hbm,
                      s_out, p_out, o_out, sidx, pidx, oidx, *rest):
        bufs = rest[:_NBUF]
        gsem = rest[_NBUF:2 * _NBUF]
        wsem = rest[2 * _NBUF:3 * _NBUF]
        isem = rest[3 * _NBUF]
        wid = lax.axis_index("s") * 2 + lax.axis_index("c")
        base = wid * b_per_w

        ih = [pltpu.async_copy(src.at[pl.ds(base, b_per_w)], dst, isem)
              for src, dst in ((ss_hbm, sidx), (ps_hbm, pidx), (os_hbm, oidx))]
        for h in ih:
            h.wait()

        items = []
        for idxr, tab, out in ((sidx, ent_hbm, s_out),
                               (pidx, rel_hbm, p_out),
                               (oidx, ent_hbm, o_out)):
            for c in range(0, b_per_w, _CHUNK):
                items.append((idxr, c, tab, out))

        g_h = [None] * _NBUF
        w_h = [None] * _NBUF
        prev = None

        def _start_writeback(i, j):
            _, off, _, out = items[i]
            g_h[j].wait()
            w_h[j] = pltpu.async_copy(
                bufs[j], out.at[pl.ds(base + off, _CHUNK)], wsem[j])

        for i, (idxr, off, tab, _) in enumerate(items):
            j = i % _NBUF
            if w_h[j] is not None:
                w_h[j].wait()
                w_h[j] = None
            g_h[j] = pltpu.async_copy(
                tab.at[idxr.at[pl.ds(off, _CHUNK)]], bufs[j], gsem[j])
            if prev is not None:
                _start_writeback(*prev)
            prev = (i, j)
        _start_writeback(*prev)
        for j in range(_NBUF):
            if w_h[j] is not None:
                w_h[j].wait()

    return gather_kernel(ss, ps, os_idx, ent, rel)


def _score_block(s_ref, p_ref, o_ref, out_ref):
    def _norm(x):
        n = jnp.sqrt(jnp.sum(x * x, axis=-1, keepdims=True))
        return x / jnp.maximum(n, 1e-12)

    d = (_norm(o_ref[...]) - _norm(s_ref[...])) - _norm(p_ref[...]) + 1e-6
    out_ref[...] = jnp.sqrt(jnp.sum(d * d, axis=-1))


def _tc_score(s_rows, p_rows, o_rows):
    batch = s_rows.shape[0]
    blk = 2048
    row_spec = pl.BlockSpec((blk, _EMBED), lambda i: (i, 0))
    return pl.pallas_call(
        _score_block,
        grid=(batch // blk,),
        in_specs=[row_spec, row_spec, row_spec],
        out_specs=pl.BlockSpec((blk,), lambda i: (i,)),
        out_shape=jax.ShapeDtypeStruct((batch,), jnp.float32),
    )(s_rows, p_rows, o_rows)


def kernel(ss, ps, os, ent_embedding, rel_embedding):
    ss = ss.astype(jnp.int32)
    ps = ps.astype(jnp.int32)
    os_idx = os.astype(jnp.int32)
    batch = ss.shape[0]
    n_slices = 2
    sl = batch // n_slices
    scores = []
    for k in range(n_slices):
        s_rows, p_rows, o_rows = _sc_gather(
            ss[k * sl:(k + 1) * sl], ps[k * sl:(k + 1) * sl],
            os_idx[k * sl:(k + 1) * sl], ent_embedding, rel_embedding)
        scores.append(_tc_score(s_rows, p_rows, o_rows))
    return jnp.concatenate(scores)
